# baseline (speedup 1.0000x reference)
import functools

import jax
import jax.numpy as jnp
from jax import lax
from jax.experimental import pallas as pl
from jax.experimental.pallas import tpu as pltpu
from jax.experimental.pallas import tpu_sc as plsc

NC = 2
NS = 16
NW = NC * NS


def _pad8(k):
    return ((k + 7) // 8) * 8


def _sc_mesh():
    return plsc.VectorSubcoreMesh(
        core_axis_name="c", subcore_axis_name="s",
        num_cores=NC, num_subcores=NS)


def _pad_nodes(n):
    return ((n + 1 + NS * 8 - 1) // (NS * 8)) * (NS * 8)


def _make_deg_kernel(n, e, chunk):
    iters = e // NW // chunk
    np_ = _pad_nodes(n)
    rpt = np_ // NS

    @functools.partial(
        pl.kernel,
        out_type=jax.ShapeDtypeStruct((NC, np_, 128), jnp.float32),
        mesh=_sc_mesh(),
        scratch_types=[
            pltpu.VMEM_SHARED((np_, 128), jnp.float32),
            pltpu.VMEM((_pad8(iters), chunk), jnp.int32),
            pltpu.VMEM((chunk, 128), jnp.float32),
            pltpu.SemaphoreType.DMA,
        ],
    )
    def deg_kernel(dst_hbm, zeros_hbm, ones_hbm, out_hbm, deg_sh, dst_v,
                   ones_v, sem):
        c = lax.axis_index("c")
        s = lax.axis_index("s")
        wid = c * NS + s
        pltpu.sync_copy(zeros_hbm, deg_sh.at[pl.ds(s * rpt, rpt)])
        pltpu.sync_copy(ones_hbm, ones_v)
        pltpu.sync_copy(dst_hbm.at[wid], dst_v)
        plsc.subcore_barrier()

        def fire(j, carry):
            pltpu.async_copy(ones_v, deg_sh.at[dst_v.at[j]], sem, add=True)
            return carry

        lax.fori_loop(0, iters, fire, 0)

        def drain(j, carry):
            pltpu.make_async_copy(zeros_hbm.at[pl.ds(0, chunk)], ones_v,
                                  sem).wait()
            return carry

        lax.fori_loop(0, iters, drain, 0)
        plsc.subcore_barrier()
        pltpu.sync_copy(deg_sh.at[pl.ds(s * rpt, rpt)],
                        out_hbm.at[c, pl.ds(s * rpt, rpt)])

    return deg_kernel


def _make_edge_kernel(n, d, e, chunk, phases):
    iters = e // NW // chunk
    assert iters % (2 * phases) == 0
    ipp = iters // phases
    ipp_pad = _pad8(ipp + 1)
    np_ = _pad_nodes(n)
    rpt = np_ // NS

    @functools.partial(
        pl.kernel,
        out_type=jax.ShapeDtypeStruct((NC, np_, d), jnp.float32),
        mesh=_sc_mesh(),
        scratch_types=[
            pltpu.VMEM_SHARED((np_, d), jnp.float32),
            pltpu.VMEM((ipp_pad, chunk), jnp.int32),
            pltpu.VMEM((ipp, chunk), jnp.int32),
            pltpu.VMEM((chunk, d), jnp.float32),
            pltpu.VMEM((chunk, d), jnp.float32),
            pltpu.SemaphoreType.DMA,
            pltpu.SemaphoreType.DMA,
        ],
    )
    def edge_kernel(h_hbm, src_hbm, dst_hbm, zeros_hbm, out_hbm,
                    acc_sh, src_v, dst_v, rows_a, rows_b,
                    gsem_a, gsem_b):
        c = lax.axis_index("c")
        s = lax.axis_index("s")
        wid = c * NS + s
        pltpu.sync_copy(zeros_hbm, acc_sh.at[pl.ds(s * rpt, rpt)])
        plsc.subcore_barrier()

        def gwait(j, buf, sem):
            pltpu.make_async_copy(h_hbm.at[src_v.at[j]], buf, sem).wait()

        for ph in range(phases):
            pltpu.sync_copy(src_hbm.at[wid, ph], src_v)
            pltpu.sync_copy(dst_hbm.at[wid, ph], dst_v)
            pltpu.async_copy(h_hbm.at[src_v.at[0]], rows_a, gsem_a)

            def pair(j, carry):
                c0 = 2 * j
                gwait(c0, rows_a, gsem_a)
                pltpu.async_copy(h_hbm.at[src_v.at[c0 + 1]], rows_b, gsem_b)
                pltpu.sync_copy(rows_a, acc_sh.at[dst_v.at[c0]], add=True)
                gwait(c0 + 1, rows_b, gsem_b)
                pltpu.async_copy(h_hbm.at[src_v.at[c0 + 2]], rows_a, gsem_a)
                pltpu.sync_copy(rows_b, acc_sh.at[dst_v.at[c0 + 1]], add=True)
                return carry

            lax.fori_loop(0, ipp // 2, pair, 0)
            gwait(ipp, rows_a, gsem_a)
        plsc.subcore_barrier()
        pltpu.sync_copy(acc_sh.at[pl.ds(s * rpt, rpt)],
                        out_hbm.at[c, pl.ds(s * rpt, rpt)])

    return edge_kernel


def _dense_pre(degp, x, W1, bn):
    n, d = x.shape

    def body(degp_ref, x_ref, w_ref, h_ref, dinv_ref):
        p = degp_ref[...]
        dv = lax.rsqrt(1.0 + p[0, :, :1] + p[1, :, :1])
        h = jnp.dot(x_ref[...], w_ref[...], preferred_element_type=jnp.float32)
        h_ref[...] = h * dv
        dinv_ref[...] = dv

    return pl.pallas_call(
        body,
        grid=(n // bn,),
        in_specs=[
            pl.BlockSpec((NC, bn, 128), lambda i: (0, i, 0)),
            pl.BlockSpec((bn, d), lambda i: (i, 0)),
            pl.BlockSpec((d, d), lambda i: (0, 0)),
        ],
        out_specs=[
            pl.BlockSpec((bn, d), lambda i: (i, 0)),
            pl.BlockSpec((bn, 1), lambda i: (i, 0)),
        ],
        out_shape=[
            jax.ShapeDtypeStruct((n, d), jnp.float32),
            jax.ShapeDtypeStruct((n, 1), jnp.float32),
        ],
    )(degp, x, W1)


def _dense_mid(accp, h1s, dinv, b1, W2, bn):
    n, d = h1s.shape

    def body(accp_ref, h_ref, dinv_ref, b_ref, w_ref, o_ref):
        p = accp_ref[...]
        dv = dinv_ref[...]
        z = (p[0] + p[1] + h_ref[...]) * dv + b_ref[...]
        g = jax.nn.gelu(z)
        o_ref[...] = jnp.dot(g, w_ref[...],
                             preferred_element_type=jnp.float32) * dv

    return pl.pallas_call(
        body,
        grid=(n // bn,),
        in_specs=[
            pl.BlockSpec((NC, bn, d), lambda i: (0, i, 0)),
            pl.BlockSpec((bn, d), lambda i: (i, 0)),
            pl.BlockSpec((bn, 1), lambda i: (i, 0)),
            pl.BlockSpec((1, d), lambda i: (0, 0)),
            pl.BlockSpec((d, d), lambda i: (0, 0)),
        ],
        out_specs=pl.BlockSpec((bn, d), lambda i: (i, 0)),
        out_shape=jax.ShapeDtypeStruct((n, d), jnp.float32),
    )(accp, h1s, dinv, b1, W2)


def _dense_post(accp, h2s, dinv, b2, bn):
    n, d = h2s.shape

    def body(accp_ref, h_ref, dinv_ref, b_ref, o_ref):
        p = accp_ref[...]
        o_ref[...] = (p[0] + p[1] + h_ref[...]) * dinv_ref[...] + b_ref[...]

    return pl.pallas_call(
        body,
        grid=(n // bn,),
        in_specs=[
            pl.BlockSpec((NC, bn, d), lambda i: (0, i, 0)),
            pl.BlockSpec((bn, d), lambda i: (i, 0)),
            pl.BlockSpec((bn, 1), lambda i: (i, 0)),
            pl.BlockSpec((1, d), lambda i: (0, 0)),
        ],
        out_specs=pl.BlockSpec((bn, d), lambda i: (i, 0)),
        out_shape=jax.ShapeDtypeStruct((n, d), jnp.float32),
    )(accp, h2s, dinv, b2)


def kernel(x, edge_index, W1, b1, W2, b2):
    n, d = x.shape
    e = edge_index.shape[1]
    chunk = 128
    phases = 2
    bn = 1000
    assert n % NS == 0 and n % bn == 0
    step = 2 * phases
    iters = -(-e // (NW * chunk))
    iters = ((iters + step - 1) // step) * step
    ipp = iters // phases
    e_pad = NW * iters * chunk

    src = edge_index[0].astype(jnp.int32)
    dst = edge_index[1].astype(jnp.int32)
    src_p = jnp.concatenate(
        [src, jnp.zeros((e_pad - e,), jnp.int32)])
    dst_p = jnp.concatenate(
        [dst, jnp.full((e_pad - e,), n, jnp.int32)])
    src4 = src_p.reshape(NW, phases, ipp, chunk)
    src4 = jnp.concatenate(
        [src4,
         jnp.zeros((NW, phases, _pad8(ipp + 1) - ipp, chunk), jnp.int32)],
        axis=2)
    dst4 = dst_p.reshape(NW, phases, ipp, chunk)
    dst3 = dst_p.reshape(NW, iters, chunk)
    rpt = _pad_nodes(n) // NS
    zeros128 = jnp.zeros((rpt, 128), jnp.float32)
    ones128 = jnp.ones((chunk, 128), jnp.float32)
    zerosd = jnp.zeros((rpt, d), jnp.float32)
    b1r = b1.reshape(1, d)
    b2r = b2.reshape(1, d)

    edge_k = _make_edge_kernel(n, d, e_pad, chunk, phases)

    degp = _make_deg_kernel(n, e_pad, chunk)(dst3, zeros128, ones128)
    h1s, dinv = _dense_pre(degp, x, W1, bn)
    acc1 = edge_k(h1s, src4, dst4, zerosd)
    h2s = _dense_mid(acc1, h1s, dinv, b1r, W2, bn)
    acc2 = edge_k(h2s, src4, dst4, zerosd)
    return _dense_post(acc2, h2s, dinv, b2r, bn)


# speedup vs baseline: 1.4855x; 1.4855x over previous
import functools

import jax
import jax.numpy as jnp
from jax import lax
from jax.experimental import pallas as pl
from jax.experimental.pallas import tpu as pltpu
from jax.experimental.pallas import tpu_sc as plsc

NC = 2
NS = 16
NW = NC * NS


def _pad8(k):
    return ((k + 7) // 8) * 8


def _sc_mesh():
    return plsc.VectorSubcoreMesh(
        core_axis_name="c", subcore_axis_name="s",
        num_cores=NC, num_subcores=NS)


def _pad_nodes(n):
    return ((n + 1 + NS * 8 - 1) // (NS * 8)) * (NS * 8)


def _make_deg_kernel(n, e, chunk):
    iters = e // NW // chunk
    np_ = _pad_nodes(n)
    rpt = np_ // NS

    @functools.partial(
        pl.kernel,
        out_type=jax.ShapeDtypeStruct((NC, np_, 128), jnp.float32),
        mesh=_sc_mesh(),
        scratch_types=[
            pltpu.VMEM_SHARED((np_, 128), jnp.float32),
            pltpu.VMEM((_pad8(iters), chunk), jnp.int32),
            pltpu.VMEM((chunk, 128), jnp.float32),
            pltpu.SemaphoreType.DMA,
        ],
    )
    def deg_kernel(dst_hbm, zeros_hbm, ones_hbm, out_hbm, deg_sh, dst_v,
                   ones_v, sem):
        c = lax.axis_index("c")
        s = lax.axis_index("s")
        wid = c * NS + s
        pltpu.sync_copy(zeros_hbm, deg_sh.at[pl.ds(s * rpt, rpt)])
        pltpu.sync_copy(ones_hbm, ones_v)
        pltpu.sync_copy(dst_hbm.at[wid], dst_v)
        plsc.subcore_barrier()

        def fire(j, carry):
            pltpu.async_copy(ones_v, deg_sh.at[dst_v.at[j]], sem, add=True)
            return carry

        lax.fori_loop(0, iters, fire, 0)

        def drain(j, carry):
            pltpu.make_async_copy(zeros_hbm.at[pl.ds(0, chunk)], ones_v,
                                  sem).wait()
            return carry

        lax.fori_loop(0, iters, drain, 0)
        plsc.subcore_barrier()
        pltpu.sync_copy(deg_sh.at[pl.ds(s * rpt, rpt)],
                        out_hbm.at[c, pl.ds(s * rpt, rpt)])

    return deg_kernel


def _make_edge_kernel(n, d, e, chunk):
    iters = e // NW // chunk
    assert iters % 2 == 0
    ept = e // NW
    np_ = _pad_nodes(n)
    rpt = np_ // NS

    @functools.partial(
        pl.kernel,
        out_type=jax.ShapeDtypeStruct((NC, np_, d), jnp.float32),
        mesh=_sc_mesh(),
        scratch_types=[
            pltpu.VMEM_SHARED((np_, d), jnp.float32),
            pltpu.VMEM((chunk,), jnp.int32),
            pltpu.VMEM((chunk,), jnp.int32),
            pltpu.VMEM((chunk,), jnp.int32),
            pltpu.VMEM((chunk,), jnp.int32),
            pltpu.VMEM((chunk, d), jnp.float32),
            pltpu.VMEM((chunk, d), jnp.float32),
            pltpu.SemaphoreType.DMA,
            pltpu.SemaphoreType.DMA,
        ],
    )
    def edge_kernel(h_hbm, src_hbm, dst_hbm, zeros_hbm, out_hbm,
                    acc_sh, src_a, dst_a, src_b, dst_b, rows_a, rows_b,
                    gsem_a, gsem_b):
        c = lax.axis_index("c")
        s = lax.axis_index("s")
        wid = c * NS + s
        base = wid * ept
        pltpu.sync_copy(zeros_hbm, acc_sh.at[pl.ds(s * rpt, rpt)])
        plsc.subcore_barrier()

        def gwait(idx, buf, sem):
            pltpu.make_async_copy(h_hbm.at[idx], buf, sem).wait()

        pltpu.sync_copy(src_hbm.at[pl.ds(base, chunk)], src_a)
        pltpu.async_copy(h_hbm.at[src_a], rows_a, gsem_a)

        def pair(j, carry):
            c0 = 2 * j
            pltpu.sync_copy(src_hbm.at[pl.ds(base + (c0 + 1) * chunk, chunk)],
                            src_b)
            pltpu.async_copy(h_hbm.at[src_b], rows_b, gsem_b)
            pltpu.sync_copy(dst_hbm.at[pl.ds(base + c0 * chunk, chunk)],
                            dst_a)
            gwait(src_a, rows_a, gsem_a)
            pltpu.sync_copy(rows_a, acc_sh.at[dst_a], add=True)
            pltpu.sync_copy(src_hbm.at[pl.ds(base + (c0 + 2) * chunk, chunk)],
                            src_a)
            pltpu.async_copy(h_hbm.at[src_a], rows_a, gsem_a)
            pltpu.sync_copy(dst_hbm.at[pl.ds(base + (c0 + 1) * chunk, chunk)],
                            dst_b)
            gwait(src_b, rows_b, gsem_b)
            pltpu.sync_copy(rows_b, acc_sh.at[dst_b], add=True)
            return carry

        lax.fori_loop(0, iters // 2, pair, 0)
        gwait(src_a, rows_a, gsem_a)
        plsc.subcore_barrier()
        pltpu.sync_copy(acc_sh.at[pl.ds(s * rpt, rpt)],
                        out_hbm.at[c, pl.ds(s * rpt, rpt)])

    return edge_kernel


def _dense_pre(degp, x, W1, bn):
    n, d = x.shape

    def body(degp_ref, x_ref, w_ref, h_ref, dinv_ref):
        p = degp_ref[...]
        dv = lax.rsqrt(1.0 + p[0, :, :1] + p[1, :, :1])
        h = jnp.dot(x_ref[...], w_ref[...], preferred_element_type=jnp.float32)
        h_ref[...] = h * dv
        dinv_ref[...] = dv

    return pl.pallas_call(
        body,
        grid=(n // bn,),
        in_specs=[
            pl.BlockSpec((NC, bn, 128), lambda i: (0, i, 0)),
            pl.BlockSpec((bn, d), lambda i: (i, 0)),
            pl.BlockSpec((d, d), lambda i: (0, 0)),
        ],
        out_specs=[
            pl.BlockSpec((bn, d), lambda i: (i, 0)),
            pl.BlockSpec((bn, 1), lambda i: (i, 0)),
        ],
        out_shape=[
            jax.ShapeDtypeStruct((n, d), jnp.float32),
            jax.ShapeDtypeStruct((n, 1), jnp.float32),
        ],
    )(degp, x, W1)


def _dense_mid(accp, h1s, dinv, b1, W2, bn):
    n, d = h1s.shape

    def body(accp_ref, h_ref, dinv_ref, b_ref, w_ref, o_ref):
        p = accp_ref[...]
        dv = dinv_ref[...]
        z = (p[0] + p[1] + h_ref[...]) * dv + b_ref[...]
        g = jax.nn.gelu(z)
        o_ref[...] = jnp.dot(g, w_ref[...],
                             preferred_element_type=jnp.float32) * dv

    return pl.pallas_call(
        body,
        grid=(n // bn,),
        in_specs=[
            pl.BlockSpec((NC, bn, d), lambda i: (0, i, 0)),
            pl.BlockSpec((bn, d), lambda i: (i, 0)),
            pl.BlockSpec((bn, 1), lambda i: (i, 0)),
            pl.BlockSpec((1, d), lambda i: (0, 0)),
            pl.BlockSpec((d, d), lambda i: (0, 0)),
        ],
        out_specs=pl.BlockSpec((bn, d), lambda i: (i, 0)),
        out_shape=jax.ShapeDtypeStruct((n, d), jnp.float32),
    )(accp, h1s, dinv, b1, W2)


def _dense_post(accp, h2s, dinv, b2, bn):
    n, d = h2s.shape

    def body(accp_ref, h_ref, dinv_ref, b_ref, o_ref):
        p = accp_ref[...]
        o_ref[...] = (p[0] + p[1] + h_ref[...]) * dinv_ref[...] + b_ref[...]

    return pl.pallas_call(
        body,
        grid=(n // bn,),
        in_specs=[
            pl.BlockSpec((NC, bn, d), lambda i: (0, i, 0)),
            pl.BlockSpec((bn, d), lambda i: (i, 0)),
            pl.BlockSpec((bn, 1), lambda i: (i, 0)),
            pl.BlockSpec((1, d), lambda i: (0, 0)),
        ],
        out_specs=pl.BlockSpec((bn, d), lambda i: (i, 0)),
        out_shape=jax.ShapeDtypeStruct((n, d), jnp.float32),
    )(accp, h2s, dinv, b2)


def kernel(x, edge_index, W1, b1, W2, b2):
    n, d = x.shape
    e = edge_index.shape[1]
    chunk = 128
    bn = 1000
    assert n % NS == 0 and n % bn == 0
    iters = -(-e // (NW * chunk))
    iters = _pad8(iters)
    e_pad = NW * iters * chunk

    src = edge_index[0].astype(jnp.int32)
    dst = edge_index[1].astype(jnp.int32)
    src1 = jnp.concatenate(
        [src, jnp.zeros((e_pad - e + chunk,), jnp.int32)])
    dst1 = jnp.concatenate(
        [dst, jnp.full((e_pad - e,), n, jnp.int32)])
    dst3 = dst1.reshape(NW, iters, chunk)
    rpt = _pad_nodes(n) // NS
    zeros128 = jnp.zeros((rpt, 128), jnp.float32)
    ones128 = jnp.ones((chunk, 128), jnp.float32)
    zerosd = jnp.zeros((rpt, d), jnp.float32)
    b1r = b1.reshape(1, d)
    b2r = b2.reshape(1, d)

    edge_k = _make_edge_kernel(n, d, e_pad, chunk)

    degp = _make_deg_kernel(n, e_pad, chunk)(dst3, zeros128, ones128)
    h1s, dinv = _dense_pre(degp, x, W1, bn)
    acc1 = edge_k(h1s, src1, dst1, zerosd)
    h2s = _dense_mid(acc1, h1s, dinv, b1r, W2, bn)
    acc2 = edge_k(h2s, src1, dst1, zerosd)
    return _dense_post(acc2, h2s, dinv, b2r, bn)


# speedup vs baseline: 1.6385x; 1.1031x over previous
import functools

import jax
import jax.numpy as jnp
from jax import lax
from jax.experimental import pallas as pl
from jax.experimental.pallas import tpu as pltpu
from jax.experimental.pallas import tpu_sc as plsc

NC = 2
NS = 16
NW = NC * NS


def _pad8(k):
    return ((k + 7) // 8) * 8


def _sc_mesh():
    return plsc.VectorSubcoreMesh(
        core_axis_name="c", subcore_axis_name="s",
        num_cores=NC, num_subcores=NS)


def _pad_nodes(n):
    return ((n + 1 + NS * 8 - 1) // (NS * 8)) * (NS * 8)


def _make_deg_kernel(n, e, chunk):
    iters = e // NW // chunk
    np_ = _pad_nodes(n)
    rpt = np_ // NS

    @functools.partial(
        pl.kernel,
        out_type=jax.ShapeDtypeStruct((NC, np_, 128), jnp.float32),
        mesh=_sc_mesh(),
        scratch_types=[
            pltpu.VMEM_SHARED((np_, 128), jnp.float32),
            pltpu.VMEM((_pad8(iters), chunk), jnp.int32),
            pltpu.VMEM((chunk, 128), jnp.float32),
            pltpu.SemaphoreType.DMA,
        ],
    )
    def deg_kernel(dst_hbm, zeros_hbm, ones_hbm, out_hbm, deg_sh, dst_v,
                   ones_v, sem):
        c = lax.axis_index("c")
        s = lax.axis_index("s")
        wid = c * NS + s
        pltpu.sync_copy(zeros_hbm, deg_sh.at[pl.ds(s * rpt, rpt)])
        pltpu.sync_copy(ones_hbm, ones_v)
        pltpu.sync_copy(dst_hbm.at[wid], dst_v)
        plsc.subcore_barrier()

        def fire(j, carry):
            pltpu.async_copy(ones_v, deg_sh.at[dst_v.at[j]], sem, add=True)
            return carry

        lax.fori_loop(0, iters, fire, 0)

        def drain(j, carry):
            pltpu.make_async_copy(zeros_hbm.at[pl.ds(0, chunk)], ones_v,
                                  sem).wait()
            return carry

        lax.fori_loop(0, iters, drain, 0)
        plsc.subcore_barrier()
        pltpu.sync_copy(deg_sh.at[pl.ds(s * rpt, rpt)],
                        out_hbm.at[c, pl.ds(s * rpt, rpt)])

    return deg_kernel


def _make_edge_kernel(n, d, e, chunk):
    iters = e // NW // chunk
    assert iters % 2 == 0
    ept = e // NW
    np_ = _pad_nodes(n)
    rpt = np_ // NS

    @functools.partial(
        pl.kernel,
        out_type=jax.ShapeDtypeStruct((NC, np_, d), jnp.float32),
        mesh=_sc_mesh(),
        scratch_types=[
            pltpu.VMEM_SHARED((np_, d), jnp.float32),
            pltpu.VMEM((chunk,), jnp.int32),
            pltpu.VMEM((chunk,), jnp.int32),
            pltpu.VMEM((chunk,), jnp.int32),
            pltpu.VMEM((chunk,), jnp.int32),
            pltpu.VMEM((chunk, d), jnp.float32),
            pltpu.VMEM((chunk, d), jnp.float32),
            pltpu.SemaphoreType.DMA,
            pltpu.SemaphoreType.DMA,
        ],
    )
    def edge_kernel(h_hbm, src_hbm, dst_hbm, zeros_hbm, out_hbm,
                    acc_sh, src_a, dst_a, src_b, dst_b, rows_a, rows_b,
                    gsem_a, gsem_b):
        c = lax.axis_index("c")
        s = lax.axis_index("s")
        wid = c * NS + s
        base = wid * ept
        pltpu.sync_copy(zeros_hbm, acc_sh.at[pl.ds(s * rpt, rpt)])
        plsc.subcore_barrier()

        def gwait(idx, buf, sem):
            pltpu.make_async_copy(h_hbm.at[idx], buf, sem).wait()

        pltpu.sync_copy(src_hbm.at[pl.ds(base, chunk)], src_a)
        pltpu.async_copy(h_hbm.at[src_a], rows_a, gsem_a)

        def pair(j, carry):
            c0 = 2 * j
            pltpu.sync_copy(src_hbm.at[pl.ds(base + (c0 + 1) * chunk, chunk)],
                            src_b)
            pltpu.async_copy(h_hbm.at[src_b], rows_b, gsem_b)
            pltpu.sync_copy(dst_hbm.at[pl.ds(base + c0 * chunk, chunk)],
                            dst_a)
            gwait(src_a, rows_a, gsem_a)
            pltpu.sync_copy(rows_a, acc_sh.at[dst_a], add=True)
            pltpu.sync_copy(src_hbm.at[pl.ds(base + (c0 + 2) * chunk, chunk)],
                            src_a)
            pltpu.async_copy(h_hbm.at[src_a], rows_a, gsem_a)
            pltpu.sync_copy(dst_hbm.at[pl.ds(base + (c0 + 1) * chunk, chunk)],
                            dst_b)
            gwait(src_b, rows_b, gsem_b)
            pltpu.sync_copy(rows_b, acc_sh.at[dst_b], add=True)
            return carry

        lax.fori_loop(0, iters // 2, pair, 0)
        gwait(src_a, rows_a, gsem_a)
        plsc.subcore_barrier()
        pltpu.sync_copy(acc_sh.at[pl.ds(s * rpt, rpt)],
                        out_hbm.at[c, pl.ds(s * rpt, rpt)])

    return edge_kernel


def _dense_pre(degp, x, W1, bn):
    n, d = x.shape

    def body(degp_ref, x_ref, w_ref, h_ref, dinv_ref):
        p = degp_ref[...]
        dv = lax.rsqrt(1.0 + p[0, :, :1] + p[1, :, :1])
        h = jnp.dot(x_ref[...], w_ref[...], preferred_element_type=jnp.float32)
        h_ref[...] = h * dv
        dinv_ref[...] = dv

    return pl.pallas_call(
        body,
        grid=(n // bn,),
        in_specs=[
            pl.BlockSpec((NC, bn, 128), lambda i: (0, i, 0)),
            pl.BlockSpec((bn, d), lambda i: (i, 0)),
            pl.BlockSpec((d, d), lambda i: (0, 0)),
        ],
        out_specs=[
            pl.BlockSpec((bn, d), lambda i: (i, 0)),
            pl.BlockSpec((bn, 1), lambda i: (i, 0)),
        ],
        out_shape=[
            jax.ShapeDtypeStruct((n, d), jnp.float32),
            jax.ShapeDtypeStruct((n, 1), jnp.float32),
        ],
    )(degp, x, W1)


def _dense_mid(accp, h1s, dinv, b1, W2, bn):
    n, d = h1s.shape

    def body(accp_ref, h_ref, dinv_ref, b_ref, w_ref, o_ref):
        p = accp_ref[...]
        dv = dinv_ref[...]
        z = (p[0] + p[1] + h_ref[...]) * dv + b_ref[...]
        g = jax.nn.gelu(z)
        o_ref[...] = jnp.dot(g, w_ref[...],
                             preferred_element_type=jnp.float32) * dv

    return pl.pallas_call(
        body,
        grid=(n // bn,),
        in_specs=[
            pl.BlockSpec((NC, bn, d), lambda i: (0, i, 0)),
            pl.BlockSpec((bn, d), lambda i: (i, 0)),
            pl.BlockSpec((bn, 1), lambda i: (i, 0)),
            pl.BlockSpec((1, d), lambda i: (0, 0)),
            pl.BlockSpec((d, d), lambda i: (0, 0)),
        ],
        out_specs=pl.BlockSpec((bn, d), lambda i: (i, 0)),
        out_shape=jax.ShapeDtypeStruct((n, d), jnp.float32),
    )(accp, h1s, dinv, b1, W2)


def _dense_post(accp, h2s, dinv, b2, bn):
    n, d = h2s.shape

    def body(accp_ref, h_ref, dinv_ref, b_ref, o_ref):
        p = accp_ref[...]
        o_ref[...] = (p[0] + p[1] + h_ref[...]) * dinv_ref[...] + b_ref[...]

    return pl.pallas_call(
        body,
        grid=(n // bn,),
        in_specs=[
            pl.BlockSpec((NC, bn, d), lambda i: (0, i, 0)),
            pl.BlockSpec((bn, d), lambda i: (i, 0)),
            pl.BlockSpec((bn, 1), lambda i: (i, 0)),
            pl.BlockSpec((1, d), lambda i: (0, 0)),
        ],
        out_specs=pl.BlockSpec((bn, d), lambda i: (i, 0)),
        out_shape=jax.ShapeDtypeStruct((n, d), jnp.float32),
    )(accp, h2s, dinv, b2)


def kernel(x, edge_index, W1, b1, W2, b2):
    n, d = x.shape
    e = edge_index.shape[1]
    chunk = 80
    bn = 1000
    assert n % NS == 0 and n % bn == 0
    iters = -(-e // (NW * chunk))
    iters = _pad8(iters)
    e_pad = NW * iters * chunk

    src = edge_index[0].astype(jnp.int32)
    dst = edge_index[1].astype(jnp.int32)
    src1 = jnp.concatenate(
        [src, jnp.zeros((e_pad - e + chunk,), jnp.int32)])
    dst1 = jnp.concatenate(
        [dst, jnp.full((e_pad - e,), n, jnp.int32)])
    dst3 = dst1.reshape(NW, iters, chunk)
    rpt = _pad_nodes(n) // NS
    zeros128 = jnp.zeros((rpt, 128), jnp.float32)
    ones128 = jnp.ones((chunk, 128), jnp.float32)
    zerosd = jnp.zeros((rpt, d), jnp.float32)
    b1r = b1.reshape(1, d)
    b2r = b2.reshape(1, d)

    edge_k = _make_edge_kernel(n, d, e_pad, chunk)

    degp = _make_deg_kernel(n, e_pad, chunk)(dst3, zeros128, ones128)
    h1s, dinv = _dense_pre(degp, x, W1, bn)
    acc1 = edge_k(h1s, src1, dst1, zerosd)
    h2s = _dense_mid(acc1, h1s, dinv, b1r, W2, bn)
    acc2 = edge_k(h2s, src1, dst1, zerosd)
    return _dense_post(acc2, h2s, dinv, b2r, bn)
